# baseline (speedup 1.0000x reference)
import functools

import jax
import jax.numpy as jnp
from jax import lax
from jax.experimental import pallas as pl
from jax.experimental.pallas import tpu as pltpu
from jax.experimental.pallas import tpu_sc as plsc

BATCH = 16384
VOCAB = 1000000
DIM = 12
EPS = 1e-5

NTILES = 16
ROWS_PER_TILE = BATCH // NTILES
CHUNK = 128
NCHUNK = ROWS_PER_TILE // CHUNK
ELEMS = ROWS_PER_TILE * DIM
GROUPS = ELEMS // 48


def _body(x_hbm, table_hbm, gamma_hbm, beta_hbm, out_hbm,
          idx_v, rows_v, stage_v, allstage_v, gb_v, fvec_v, shared_s, sem):
    sid = lax.axis_index("s")
    base = sid * ROWS_PER_TILE

    copies = []
    for j in range(NCHUNK):
        pltpu.sync_copy(x_hbm.at[pl.ds(base + j * CHUNK, CHUNK)], idx_v.at[j])
        copies.append(
            pltpu.async_copy(table_hbm.at[idx_v.at[j]],
                             rows_v.at[pl.ds(j * CHUNK, CHUNK)], sem))
    pltpu.sync_copy(gamma_hbm, gb_v.at[0])
    pltpu.sync_copy(beta_hbm, gb_v.at[1])
    for c in copies:
        c.wait()

    iota = lax.iota(jnp.int32, 16)
    v12 = jnp.full((16,), DIM, jnp.int32)
    v16 = jnp.full((16,), 16, jnp.int32)
    v48 = jnp.full((16,), 48, jnp.int32)
    rbase = []
    cbase = []
    for k in range(3):
        fl = iota + (16 * k)
        r = lax.div(fl, v12)
        rbase.append(r)
        cbase.append(fl - r * DIM)

    zero = jnp.zeros((16,), jnp.float32)

    def stats_body(g, carry):
        accs = list(carry)
        roff = 4 * g
        for k in range(3):
            v = plsc.load_gather(rows_v, [rbase[k] + roff, cbase[k]])
            accs[k] = accs[k] + v
            accs[3 + k] = accs[3 + k] + v * v
        return tuple(accs)

    accs = lax.fori_loop(0, GROUPS, stats_body, (zero,) * 6, unroll=2)

    for k in range(6):
        stage_v[k, :] = accs[k]

    pltpu.sync_copy(stage_v, shared_s.at[sid])
    plsc.subcore_barrier()
    pltpu.sync_copy(shared_s, allstage_v)

    tot = [zero] * 6
    for t in range(NTILES):
        for k in range(6):
            tot[k] = tot[k] + allstage_v[t, k, :]
    for k in range(6):
        stage_v[k, :] = tot[k]

    featsum = zero
    featsq = zero
    for o in range(4):
        pos = lax.rem(iota + 12 * o, v48)
        kk = lax.div(pos, v16)
        ll = pos - kk * 16
        featsum = featsum + plsc.load_gather(stage_v, [kk, ll])
        featsq = featsq + plsc.load_gather(stage_v, [kk + 3, ll])

    inv_n = jnp.float32(1.0 / BATCH)
    mean = featsum * inv_n
    var = jnp.maximum(featsq * inv_n - mean * mean, 0.0)

    t = var + jnp.float32(EPS)
    seed_i = jnp.int32(0x5F3759DF) - lax.shift_right_logical(
        plsc.bitcast(t, jnp.int32), 1)
    y = plsc.bitcast(seed_i, jnp.float32)
    half_t = t * jnp.float32(0.5)
    for _ in range(3):
        y = y * (jnp.float32(1.5) - half_t * y * y)

    scale = y * gb_v[0, :]
    shift = gb_v[1, :] - mean * scale
    fvec_v[0, :] = scale
    fvec_v[1, :] = shift

    zeros16 = jnp.zeros((16,), jnp.int32)
    scale_k = [plsc.load_gather(fvec_v, [zeros16, cbase[k]]) for k in range(3)]
    shift_k = [plsc.load_gather(fvec_v, [zeros16 + 1, cbase[k]]) for k in range(3)]

    def norm_body(g, carry):
        roff = 4 * g
        for k in range(3):
            r = rbase[k] + roff
            v = plsc.load_gather(rows_v, [r, cbase[k]])
            plsc.store_scatter(rows_v, [r, cbase[k]], v * scale_k[k] + shift_k[k])
        return carry

    lax.fori_loop(0, GROUPS, norm_body, 0, unroll=2)

    pltpu.sync_copy(rows_v, out_hbm.at[pl.ds(base, ROWS_PER_TILE)])


@jax.jit
def kernel(x, table, gamma, beta):
    x = x.astype(jnp.int32)
    gamma16 = jnp.pad(gamma.astype(jnp.float32), (0, 16 - DIM))
    beta16 = jnp.pad(beta.astype(jnp.float32), (0, 16 - DIM))

    mesh = plsc.VectorSubcoreMesh(
        core_axis_name="c", subcore_axis_name="s", num_cores=1)
    k = pl.kernel(
        _body,
        out_type=jax.ShapeDtypeStruct((BATCH, DIM), jnp.float32),
        mesh=mesh,
        scratch_types=[
            pltpu.VMEM((NCHUNK, CHUNK), jnp.int32),
            pltpu.VMEM((ROWS_PER_TILE, DIM), jnp.float32),
            pltpu.VMEM((6, 16), jnp.float32),
            pltpu.VMEM((NTILES, 6, 16), jnp.float32),
            pltpu.VMEM((2, 16), jnp.float32),
            pltpu.VMEM((2, 16), jnp.float32),
            pltpu.VMEM_SHARED((NTILES, 6, 16), jnp.float32),
            pltpu.SemaphoreType.DMA,
        ],
        compiler_params=pltpu.CompilerParams(
            use_tc_tiling_on_sc=False, needs_layout_passes=False),
    )
    return k(x, table, gamma16, beta16)


# speedup vs baseline: 1.4693x; 1.4693x over previous
import jax
import jax.numpy as jnp
from jax import lax
from jax.experimental import pallas as pl
from jax.experimental.pallas import tpu as pltpu
from jax.experimental.pallas import tpu_sc as plsc

BATCH = 16384
VOCAB = 1000000
DIM = 12
EPS = 1e-5

NCORES = 2
NSUB = 16
NW = NCORES * NSUB
RPT = BATCH // NW


def _gather_body(x_hbm, table_hbm, e_hbm, idx_v, sem):
    cid = lax.axis_index("c")
    sid = lax.axis_index("s")
    wid = cid * NSUB + sid
    base = wid * RPT

    pltpu.sync_copy(x_hbm.at[pl.ds(base, RPT)], idx_v)

    def fire(g, carry):
        ivec = idx_v[pl.ds(16 * g, 16)]
        for l in range(16):
            xi = lax.squeeze(lax.slice(ivec, (l,), (l + 1,)), (0,))
            pltpu.make_async_copy(
                table_hbm.at[pl.ds(xi, 1)],
                e_hbm.at[pl.ds(base + 16 * g + l, 1)],
                sem).start()
        return carry

    lax.fori_loop(0, RPT // 16, fire, 0)

    def drain(i, carry):
        pltpu.make_async_copy(
            table_hbm.at[pl.ds(0, 1)],
            e_hbm.at[pl.ds(base, 1)],
            sem).wait()
        return carry

    lax.fori_loop(0, RPT, drain, 0, unroll=8)


def _bn_body(e_ref, g_ref, b_ref, y_ref):
    e = e_ref[...]
    mean = jnp.mean(e, axis=0, keepdims=True)
    var = jnp.mean((e - mean) * (e - mean), axis=0, keepdims=True)
    inv = lax.rsqrt(var + EPS)
    y_ref[...] = (e - mean) * (inv * g_ref[...]) + b_ref[...]


@jax.jit
def kernel(x, table, gamma, beta):
    x = x.astype(jnp.int32)

    mesh = plsc.VectorSubcoreMesh(
        core_axis_name="c", subcore_axis_name="s", num_cores=NCORES)
    gather = pl.kernel(
        _gather_body,
        out_type=jax.ShapeDtypeStruct((BATCH, DIM), jnp.float32),
        mesh=mesh,
        scratch_types=[
            pltpu.VMEM((RPT,), jnp.int32),
            pltpu.SemaphoreType.DMA,
        ],
        compiler_params=pltpu.CompilerParams(
            use_tc_tiling_on_sc=True, needs_layout_passes=False),
    )
    e = gather(x, table)

    y = pl.pallas_call(
        _bn_body,
        out_shape=jax.ShapeDtypeStruct((BATCH, DIM), jnp.float32),
    )(e, gamma.reshape(1, DIM), beta.reshape(1, DIM))
    return y
